# baseline (speedup 1.0000x reference)
import jax
import jax.numpy as jnp
from jax.experimental import pallas as pl
from jax.experimental.pallas import tpu as pltpu


def _scatter_kernel(src_ref, dst_ref, h_ref, out_ref):
    i = pl.program_id(0)

    @pl.when(i == 0)
    def _():
        out_ref[...] = jnp.zeros_like(out_ref)

    def body(e, carry):
        s = src_ref[e]
        d = dst_ref[e]
        row = h_ref[pl.ds(s, 1), :]
        out_ref[pl.ds(d, 1), :] += row
        return carry

    jax.lax.fori_loop(0, src_ref.shape[0], body, 0, unroll=8)


def _message_pass(h, src, dst):
    n, d = h.shape
    e = src.shape[0]
    ec = 16384
    nb = -(-e // ec)
    e_pad = nb * ec
    src = jnp.concatenate([src, jnp.zeros((e_pad - e,), jnp.int32)])
    dst = jnp.concatenate([dst, jnp.full((e_pad - e,), n, jnp.int32)])
    out = pl.pallas_call(
        _scatter_kernel,
        grid=(nb,),
        in_specs=[
            pl.BlockSpec((ec,), lambda i: (i,), memory_space=pltpu.SMEM),
            pl.BlockSpec((ec,), lambda i: (i,), memory_space=pltpu.SMEM),
            pl.BlockSpec((n, d), lambda i: (0, 0)),
        ],
        out_specs=pl.BlockSpec((n + 8, d), lambda i: (0, 0)),
        out_shape=jax.ShapeDtypeStruct((n + 8, d), h.dtype),
    )(src, dst, h)
    return out[:n]


def _mlp_kernel(h_ref, agg_ref, wa_ref, ba_ref, wb_ref, bb_ref,
                y_ref, s_ref, ss_ref):
    i = pl.program_id(0)
    z = h_ref[...] + agg_ref[...]
    z = jnp.maximum(
        jnp.dot(z, wa_ref[...], preferred_element_type=jnp.float32)
        + ba_ref[...], 0.0)
    z = jnp.dot(z, wb_ref[...], preferred_element_type=jnp.float32) + bb_ref[...]
    z = jnp.maximum(z, 0.0)
    y_ref[...] = z

    @pl.when(i == 0)
    def _():
        s_ref[...] = jnp.zeros_like(s_ref)
        ss_ref[...] = jnp.zeros_like(ss_ref)

    s_ref[...] += jnp.sum(z, axis=0, keepdims=True)
    ss_ref[...] += jnp.sum(z * z, axis=0, keepdims=True)


def _mlp(h, agg, wa, ba, wb, bb):
    n, d_in = h.shape
    dim = wa.shape[1]
    rb = 2000
    nb = n // rb
    y, s, ss = pl.pallas_call(
        _mlp_kernel,
        grid=(nb,),
        in_specs=[
            pl.BlockSpec((rb, d_in), lambda i: (i, 0)),
            pl.BlockSpec((rb, d_in), lambda i: (i, 0)),
            pl.BlockSpec((d_in, dim), lambda i: (0, 0)),
            pl.BlockSpec((1, dim), lambda i: (0, 0)),
            pl.BlockSpec((dim, dim), lambda i: (0, 0)),
            pl.BlockSpec((1, dim), lambda i: (0, 0)),
        ],
        out_specs=[
            pl.BlockSpec((rb, dim), lambda i: (i, 0)),
            pl.BlockSpec((1, dim), lambda i: (0, 0)),
            pl.BlockSpec((1, dim), lambda i: (0, 0)),
        ],
        out_shape=[
            jax.ShapeDtypeStruct((n, dim), jnp.float32),
            jax.ShapeDtypeStruct((1, dim), jnp.float32),
            jax.ShapeDtypeStruct((1, dim), jnp.float32),
        ],
    )(h, agg, wa, ba.reshape(1, dim), wb, bb.reshape(1, dim))
    return y, s, ss


def _norm_kernel(y_ref, sc_ref, sh_ref, o_ref):
    o_ref[...] = y_ref[...] * sc_ref[...] + sh_ref[...]


def _norm(y, scale, shift):
    n, dim = y.shape
    rb = 2000
    nb = n // rb
    return pl.pallas_call(
        _norm_kernel,
        grid=(nb,),
        in_specs=[
            pl.BlockSpec((rb, dim), lambda i: (i, 0)),
            pl.BlockSpec((1, dim), lambda i: (0, 0)),
            pl.BlockSpec((1, dim), lambda i: (0, 0)),
        ],
        out_specs=pl.BlockSpec((rb, dim), lambda i: (i, 0)),
        out_shape=jax.ShapeDtypeStruct((n, dim), jnp.float32),
    )(y, scale.reshape(1, dim), shift.reshape(1, dim))


def _segmax_kernel(b_ref, z_ref, o_ref):
    i = pl.program_id(0)

    @pl.when(i == 0)
    def _():
        o_ref[...] = jnp.full_like(o_ref, -jnp.inf)

    def body(r, carry):
        b = b_ref[r]
        row = z_ref[pl.ds(r, 1), :]
        o_ref[pl.ds(b, 1), :] = jnp.maximum(o_ref[pl.ds(b, 1), :], row)
        return carry

    jax.lax.fori_loop(0, z_ref.shape[0], body, 0, unroll=8)


def _segmax(batch, z, g):
    n, w = z.shape
    rb = 2048
    nb = -(-n // rb)
    n_pad = nb * rb
    batch = jnp.concatenate([batch, jnp.full((n_pad - n,), g, jnp.int32)])
    z = jnp.concatenate([z, jnp.zeros((n_pad - n, w), jnp.float32)])
    out = pl.pallas_call(
        _segmax_kernel,
        grid=(nb,),
        in_specs=[
            pl.BlockSpec((rb,), lambda i: (i,), memory_space=pltpu.SMEM),
            pl.BlockSpec((rb, w), lambda i: (i, 0)),
        ],
        out_specs=pl.BlockSpec((g + 8, w), lambda i: (0, 0)),
        out_shape=jax.ShapeDtypeStruct((g + 8, w), jnp.float32),
    )(batch, z)
    return out[:g]


def kernel(x, edge_index, batch,
           w0a, b0a, w0b, b0b, g0, be0,
           w1a, b1a, w1b, b1b, g1, be1,
           w2a, b2a, w2b, b2b, g2, be2):
    src = edge_index[0]
    dst = edge_index[1]
    n = x.shape[0]
    params = [(w0a, b0a, w0b, b0b, g0, be0),
              (w1a, b1a, w1b, b1b, g1, be1),
              (w2a, b2a, w2b, b2b, g2, be2)]
    h = x
    outs = []
    for (wa, ba, wb, bb, g, be) in params:
        agg = _message_pass(h, src, dst)
        y, s, ss = _mlp(h, agg, wa, ba, wb, bb)
        mean = s[0] / n
        var = ss[0] / n - mean * mean
        scale = g * jax.lax.rsqrt(var + 1e-5)
        shift = be - mean * scale
        z = _norm(y, scale, shift)
        outs.append(z)
        h = z
    node_rep = jnp.concatenate(outs, axis=1)
    return _segmax(batch, node_rep, 256)


# speedup vs baseline: 1.0159x; 1.0159x over previous
import jax
import jax.numpy as jnp
from jax.experimental import pallas as pl
from jax.experimental.pallas import tpu as pltpu


def _scatter_kernel(src_ref, dst_ref, h_ref, out_ref):
    i = pl.program_id(0)

    @pl.when(i == 0)
    def _():
        out_ref[...] = jnp.zeros_like(out_ref)

    def body(e, carry):
        s = src_ref[e]
        d = dst_ref[e]
        row = h_ref[pl.ds(s, 1), :]
        out_ref[pl.ds(d, 1), :] += row
        return carry

    jax.lax.fori_loop(0, src_ref.shape[0], body, 0, unroll=4)


def _message_pass(h, src, dst):
    n, d = h.shape
    e = src.shape[0]
    ec = 8192
    nb = -(-e // ec)
    e_pad = nb * ec
    src = jnp.concatenate([src, jnp.zeros((e_pad - e,), jnp.int32)])
    dst = jnp.concatenate([dst, jnp.full((e_pad - e,), n, jnp.int32)])
    out = pl.pallas_call(
        _scatter_kernel,
        grid=(nb,),
        in_specs=[
            pl.BlockSpec((ec,), lambda i: (i,), memory_space=pltpu.SMEM),
            pl.BlockSpec((ec,), lambda i: (i,), memory_space=pltpu.SMEM),
            pl.BlockSpec((n, d), lambda i: (0, 0)),
        ],
        out_specs=pl.BlockSpec((n + 8, d), lambda i: (0, 0)),
        out_shape=jax.ShapeDtypeStruct((n + 8, d), h.dtype),
    )(src, dst, h)
    return out[:n]


def _mlp_kernel(h_ref, agg_ref, wa_ref, ba_ref, wb_ref, bb_ref,
                y_ref, s_ref, ss_ref):
    i = pl.program_id(0)
    z = h_ref[...] + agg_ref[...]
    z = jnp.maximum(
        jnp.dot(z, wa_ref[...], preferred_element_type=jnp.float32)
        + ba_ref[...], 0.0)
    z = jnp.dot(z, wb_ref[...], preferred_element_type=jnp.float32) + bb_ref[...]
    z = jnp.maximum(z, 0.0)
    y_ref[...] = z

    @pl.when(i == 0)
    def _():
        s_ref[...] = jnp.zeros_like(s_ref)
        ss_ref[...] = jnp.zeros_like(ss_ref)

    s_ref[...] += jnp.sum(z, axis=0, keepdims=True)
    ss_ref[...] += jnp.sum(z * z, axis=0, keepdims=True)


def _mlp(h, agg, wa, ba, wb, bb):
    n, d_in = h.shape
    dim = wa.shape[1]
    rb = 2000
    nb = n // rb
    y, s, ss = pl.pallas_call(
        _mlp_kernel,
        grid=(nb,),
        in_specs=[
            pl.BlockSpec((rb, d_in), lambda i: (i, 0)),
            pl.BlockSpec((rb, d_in), lambda i: (i, 0)),
            pl.BlockSpec((d_in, dim), lambda i: (0, 0)),
            pl.BlockSpec((1, dim), lambda i: (0, 0)),
            pl.BlockSpec((dim, dim), lambda i: (0, 0)),
            pl.BlockSpec((1, dim), lambda i: (0, 0)),
        ],
        out_specs=[
            pl.BlockSpec((rb, dim), lambda i: (i, 0)),
            pl.BlockSpec((1, dim), lambda i: (0, 0)),
            pl.BlockSpec((1, dim), lambda i: (0, 0)),
        ],
        out_shape=[
            jax.ShapeDtypeStruct((n, dim), jnp.float32),
            jax.ShapeDtypeStruct((1, dim), jnp.float32),
            jax.ShapeDtypeStruct((1, dim), jnp.float32),
        ],
    )(h, agg, wa, ba.reshape(1, dim), wb, bb.reshape(1, dim))
    return y, s, ss


def _norm_kernel(y_ref, sc_ref, sh_ref, o_ref):
    o_ref[...] = y_ref[...] * sc_ref[...] + sh_ref[...]


def _norm(y, scale, shift):
    n, dim = y.shape
    rb = 2000
    nb = n // rb
    return pl.pallas_call(
        _norm_kernel,
        grid=(nb,),
        in_specs=[
            pl.BlockSpec((rb, dim), lambda i: (i, 0)),
            pl.BlockSpec((1, dim), lambda i: (0, 0)),
            pl.BlockSpec((1, dim), lambda i: (0, 0)),
        ],
        out_specs=pl.BlockSpec((rb, dim), lambda i: (i, 0)),
        out_shape=jax.ShapeDtypeStruct((n, dim), jnp.float32),
    )(y, scale.reshape(1, dim), shift.reshape(1, dim))


def _segmax_kernel(b_ref, z_ref, o_ref):
    i = pl.program_id(0)

    @pl.when(i == 0)
    def _():
        o_ref[...] = jnp.full_like(o_ref, -jnp.inf)

    def body(r, carry):
        b = b_ref[r]
        row = z_ref[pl.ds(r, 1), :]
        o_ref[pl.ds(b, 1), :] = jnp.maximum(o_ref[pl.ds(b, 1), :], row)
        return carry

    jax.lax.fori_loop(0, z_ref.shape[0], body, 0, unroll=4)


def _segmax(batch, z, g):
    n, w = z.shape
    rb = 2048
    nb = -(-n // rb)
    n_pad = nb * rb
    batch = jnp.concatenate([batch, jnp.full((n_pad - n,), g, jnp.int32)])
    z = jnp.concatenate([z, jnp.zeros((n_pad - n, w), jnp.float32)])
    out = pl.pallas_call(
        _segmax_kernel,
        grid=(nb,),
        in_specs=[
            pl.BlockSpec((rb,), lambda i: (i,), memory_space=pltpu.SMEM),
            pl.BlockSpec((rb, w), lambda i: (i, 0)),
        ],
        out_specs=pl.BlockSpec((g + 8, w), lambda i: (0, 0)),
        out_shape=jax.ShapeDtypeStruct((g + 8, w), jnp.float32),
    )(batch, z)
    return out[:g]


def kernel(x, edge_index, batch,
           w0a, b0a, w0b, b0b, g0, be0,
           w1a, b1a, w1b, b1b, g1, be1,
           w2a, b2a, w2b, b2b, g2, be2):
    src = edge_index[0]
    dst = edge_index[1]
    n = x.shape[0]
    params = [(w0a, b0a, w0b, b0b, g0, be0),
              (w1a, b1a, w1b, b1b, g1, be1),
              (w2a, b2a, w2b, b2b, g2, be2)]
    h = x
    outs = []
    for (wa, ba, wb, bb, g, be) in params:
        agg = _message_pass(h, src, dst)
        y, s, ss = _mlp(h, agg, wa, ba, wb, bb)
        mean = s[0] / n
        var = ss[0] / n - mean * mean
        scale = g * jax.lax.rsqrt(var + 1e-5)
        shift = be - mean * scale
        z = _norm(y, scale, shift)
        outs.append(z)
        h = z
    node_rep = jnp.concatenate(outs, axis=1)
    return _segmax(batch, node_rep, 256)
